# baseline (speedup 1.0000x reference)
import jax
import jax.numpy as jnp
from jax.experimental import pallas as pl
from jax.experimental.pallas import tpu as pltpu

B = 8
MAXN = 1024
HID = 64
OUT = 16
N1 = 103
N2 = 11

_BF = jnp.float32
GPB = 4


def _mm(a, b):
    return jax.lax.dot_general(a, b, (((1,), (0,)), ((), ())),
                               preferred_element_type=jnp.float32)


def _mm_t(a, b):
    return jax.lax.dot_general(a, b, (((0,), (0,)), ((), ())),
                               preferred_element_type=jnp.float32)


def _softmax(z):
    z = z - jnp.max(z, axis=-1, keepdims=True)
    e = jnp.exp(z)
    return e * (1.0 / jnp.sum(e, axis=-1, keepdims=True))


def _diffpool_body(x_ref, adj_ref, W1p0_ref, W1p1_ref, W1e0_ref, W1e1_ref,
                   W2p0_ref, W2p1_ref, W2e0_ref, W2e1_ref, W3a_ref, W3b_ref,
                   out_ref, xp_buf, ap_buf):
    G = range(GPB)
    relu = jax.nn.relu
    step = pl.program_id(0)

    adj = [adj_ref[g].astype(_BF) for g in G]
    t = [_mm(adj[g], x_ref[g].astype(_BF)) for g in G]
    s1 = [relu(_mm(t[g], W1p0_ref[...])).astype(_BF) for g in G]
    h1 = [relu(_mm(t[g], W1e0_ref[...])).astype(_BF) for g in G]
    u = [_mm(adj[g], s1[g]) for g in G]
    v = [_mm(adj[g], h1[g]) for g in G]
    s = [relu(_mm(u[g], W1p1_ref[...])) for g in G]
    h = [relu(_mm(v[g], W1e1_ref[...])).astype(_BF) for g in G]

    ss = [_softmax(s[g]).astype(_BF) for g in G]
    w = [_mm(adj[g], ss[g]).astype(_BF) for g in G]
    for g in G:
        xp_buf[step * GPB + g] = _mm_t(ss[g], h[g])
        ap_buf[step * GPB + g] = _mm_t(ss[g], w[g])

    @pl.when(step == (B // GPB) - 1)
    def _tail():
        GA = range(B)
        x_p = [xp_buf[g] for g in GA]
        a_p = [ap_buf[g] for g in GA]

        t2 = [_mm(a_p[g], x_p[g]) for g in GA]
        s2a = [relu(_mm(t2[g], W2p0_ref[...])) for g in GA]
        h2a = [relu(_mm(t2[g], W2e0_ref[...])) for g in GA]
        u2 = [_mm(a_p[g], s2a[g]) for g in GA]
        v2 = [_mm(a_p[g], h2a[g]) for g in GA]
        s2 = [relu(_mm(u2[g], W2p1_ref[...])) for g in GA]
        h2 = [relu(_mm(v2[g], W2e1_ref[...])) for g in GA]

        ss2 = [_softmax(s2[g]) for g in GA]
        x_q = [_mm_t(ss2[g], h2[g]) for g in GA]
        w2 = [_mm(a_p[g], ss2[g]) for g in GA]
        a_q = [_mm_t(ss2[g], w2[g]) for g in GA]

        z1 = [relu(_mm(_mm(a_q[g], x_q[g]), W3a_ref[...])) for g in GA]
        z2 = [relu(_mm(_mm(a_q[g], z1[g]), W3b_ref[...])) for g in GA]
        for g in GA:
            out_ref[g, 0] = jnp.mean(z2[g], axis=0)


def kernel(x, adj, W1p0, W1p1, W1e0, W1e1, W2p0, W2p1, W2e0, W2e1, W3a, W3b):
    w_spec = lambda shp: pl.BlockSpec(shp, lambda b: (0,) * len(shp))
    out = pl.pallas_call(
        _diffpool_body,
        grid=(B // GPB,),
        in_specs=[
            pl.BlockSpec((GPB, MAXN, HID), lambda b: (b, 0, 0)),
            pl.BlockSpec((GPB, MAXN, MAXN), lambda b: (b, 0, 0)),
            w_spec(W1p0.shape), w_spec(W1p1.shape),
            w_spec(W1e0.shape), w_spec(W1e1.shape),
            w_spec(W2p0.shape), w_spec(W2p1.shape),
            w_spec(W2e0.shape), w_spec(W2e1.shape),
            w_spec(W3a.shape), w_spec(W3b.shape),
        ],
        out_specs=pl.BlockSpec((B, 1, OUT), lambda b: (0, 0, 0)),
        out_shape=jax.ShapeDtypeStruct((B, 1, OUT), jnp.float32),
        scratch_shapes=[
            pltpu.VMEM((B, N1, HID), jnp.float32),
            pltpu.VMEM((B, N1, N1), jnp.float32),
        ],
        compiler_params=pltpu.CompilerParams(
            dimension_semantics=("arbitrary",),
        ),
    )(x, adj, W1p0, W1p1, W1e0, W1e1, W2p0, W2p1, W2e0, W2e1, W3a, W3b)
    return out.reshape(B, OUT)


# speedup vs baseline: 1.0477x; 1.0477x over previous
import jax
import jax.numpy as jnp
from jax.experimental import pallas as pl
from jax.experimental.pallas import tpu as pltpu

B = 8
MAXN = 1024
HID = 64
OUT = 16
N1 = 103
N2 = 11

_BF = jnp.bfloat16
GPB = 2


def _mm(a, b):
    return jax.lax.dot_general(a, b, (((1,), (0,)), ((), ())),
                               preferred_element_type=jnp.float32)


def _mm_t(a, b):
    return jax.lax.dot_general(a, b, (((0,), (0,)), ((), ())),
                               preferred_element_type=jnp.float32)


def _softmax(z):
    z = z - jnp.max(z, axis=-1, keepdims=True)
    e = jnp.exp(z)
    return e * (1.0 / jnp.sum(e, axis=-1, keepdims=True))


def _diffpool_body(x_ref, adj_ref, W1p0_ref, W1p1_ref, W1e0_ref, W1e1_ref,
                   W2p0_ref, W2p1_ref, W2e0_ref, W2e1_ref, W3a_ref, W3b_ref,
                   out_ref, xp_buf, ap_buf):
    G = range(GPB)
    relu = jax.nn.relu
    step = pl.program_id(0)

    adj = [adj_ref[g].astype(_BF) for g in G]
    t = [_mm(adj[g], x_ref[g].astype(_BF)) for g in G]
    s1 = [relu(_mm(t[g], W1p0_ref[...])).astype(_BF) for g in G]
    h1 = [relu(_mm(t[g], W1e0_ref[...])).astype(_BF) for g in G]
    u = [_mm(adj[g], s1[g]) for g in G]
    v = [_mm(adj[g], h1[g]) for g in G]
    s = [relu(_mm(u[g], W1p1_ref[...])) for g in G]
    h = [relu(_mm(v[g], W1e1_ref[...])).astype(_BF) for g in G]

    ss = [_softmax(s[g]).astype(_BF) for g in G]
    w = [_mm(adj[g], ss[g]).astype(_BF) for g in G]
    for g in G:
        xp_buf[step * GPB + g] = _mm_t(ss[g], h[g])
        ap_buf[step * GPB + g] = _mm_t(ss[g], w[g])

    @pl.when(step == (B // GPB) - 1)
    def _tail():
        GA = range(B)
        x_p = [xp_buf[g] for g in GA]
        a_p = [ap_buf[g] for g in GA]

        t2 = [_mm(a_p[g], x_p[g]) for g in GA]
        s2a = [relu(_mm(t2[g], W2p0_ref[...])) for g in GA]
        h2a = [relu(_mm(t2[g], W2e0_ref[...])) for g in GA]
        u2 = [_mm(a_p[g], s2a[g]) for g in GA]
        v2 = [_mm(a_p[g], h2a[g]) for g in GA]
        s2 = [relu(_mm(u2[g], W2p1_ref[...])) for g in GA]
        h2 = [relu(_mm(v2[g], W2e1_ref[...])) for g in GA]

        ss2 = [_softmax(s2[g]) for g in GA]
        x_q = [_mm_t(ss2[g], h2[g]) for g in GA]
        w2 = [_mm(a_p[g], ss2[g]) for g in GA]
        a_q = [_mm_t(ss2[g], w2[g]) for g in GA]

        z1 = [relu(_mm(_mm(a_q[g], x_q[g]), W3a_ref[...])) for g in GA]
        z2 = [relu(_mm(_mm(a_q[g], z1[g]), W3b_ref[...])) for g in GA]
        for g in GA:
            out_ref[g, 0] = jnp.mean(z2[g], axis=0)


def kernel(x, adj, W1p0, W1p1, W1e0, W1e1, W2p0, W2p1, W2e0, W2e1, W3a, W3b):
    w_spec = lambda shp: pl.BlockSpec(shp, lambda b: (0,) * len(shp))
    out = pl.pallas_call(
        _diffpool_body,
        grid=(B // GPB,),
        in_specs=[
            pl.BlockSpec((GPB, MAXN, HID), lambda b: (b, 0, 0)),
            pl.BlockSpec((GPB, MAXN, MAXN), lambda b: (b, 0, 0)),
            w_spec(W1p0.shape), w_spec(W1p1.shape),
            w_spec(W1e0.shape), w_spec(W1e1.shape),
            w_spec(W2p0.shape), w_spec(W2p1.shape),
            w_spec(W2e0.shape), w_spec(W2e1.shape),
            w_spec(W3a.shape), w_spec(W3b.shape),
        ],
        out_specs=pl.BlockSpec((B, 1, OUT), lambda b: (0, 0, 0)),
        out_shape=jax.ShapeDtypeStruct((B, 1, OUT), jnp.float32),
        scratch_shapes=[
            pltpu.VMEM((B, N1, HID), jnp.float32),
            pltpu.VMEM((B, N1, N1), jnp.float32),
        ],
        compiler_params=pltpu.CompilerParams(
            dimension_semantics=("arbitrary",),
        ),
    )(x, adj, W1p0, W1p1, W1e0, W1e1, W2p0, W2p1, W2e0, W2e1, W3a, W3b)
    return out.reshape(B, OUT)
